# baseline (speedup 1.0000x reference)
import jax
import jax.numpy as jnp
from jax import lax
from jax.experimental import pallas as pl
from jax.experimental.pallas import tpu as pltpu
from jax.experimental.pallas import tpu_sc as plsc

NC, NS, L = 2, 16, 16
NW = NC * NS
B, S, D = 4096, 50, 64
BPW = B // NW
CH = 2
ROWS = CH * S
NCHUNK = BPW // CH
ND = D // L
INV_S = 1.0 / S


def _pool_body(idx_hbm, par_hbm, table_hbm, out_hbm, idx_v, par_v, buf, out_v, sem0, sem1):
    wid = lax.axis_index("c") * NS + lax.axis_index("s")
    pltpu.sync_copy(idx_hbm.at[wid], idx_v)
    pltpu.sync_copy(par_hbm.at[wid], par_v)
    sems = (sem0, sem1)

    def start(c, b):
        pltpu.async_copy(table_hbm.at[idx_v.at[c]], buf.at[b], sems[b])

    def wait(b):
        pltpu.make_async_copy(table_hbm.at[idx_v.at[0]], buf.at[b], sems[b]).wait()

    def accum(c, b):
        for e in range(CH):
            base = e * S
            acc = None
            for s in range(0, S):
                pv = par_v[c, pl.ds((base + s) // L * L, L)]
                p = pv[(base + s) % L]
                row = []
                for d in range(ND):
                    lo = buf[b, base + s, pl.ds(d * L, L)]
                    hi = buf[b, base + s, pl.ds(D + d * L, L)]
                    row.append(jnp.where(p == 1, hi, lo))
                if acc is None:
                    acc = row
                else:
                    acc = [a + r for a, r in zip(acc, row)]
            for d in range(ND):
                out_v[c * CH + e, pl.ds(d * L, L)] = acc[d] * jnp.float32(INV_S)

    start(0, 0)
    start(1, 1)

    def loop_body(t, carry):
        for b in range(2):
            c = t * 2 + b
            wait(b)
            accum(c, b)
            start(c + 2, b)
        return carry

    lax.fori_loop(0, NCHUNK // 2 - 1, loop_body, 0)
    for b in range(2):
        wait(b)
        accum(NCHUNK - 2 + b, b)

    pltpu.sync_copy(out_v, out_hbm.at[pl.ds(wid * BPW, BPW)])


def kernel(inputs, cvm, table_pri, table_sec):
    del cvm, table_sec
    idx32 = inputs.astype(jnp.int32)
    idx = (idx32 >> 1).reshape(NW, NCHUNK, ROWS)
    par = jnp.pad((idx32 & 1).reshape(NW, NCHUNK, ROWS), ((0, 0), (0, 0), (0, 12)))
    table2 = table_pri.reshape(500000, 128)
    run = pl.kernel(
        _pool_body,
        out_type=jax.ShapeDtypeStruct((B, D), jnp.float32),
        mesh=plsc.VectorSubcoreMesh(core_axis_name="c", subcore_axis_name="s"),
        scratch_types=[
            pltpu.VMEM((NCHUNK, ROWS), jnp.int32),
            pltpu.VMEM((NCHUNK, ROWS + 12), jnp.int32),
            pltpu.VMEM((2, ROWS, 128), jnp.float32),
            pltpu.VMEM((BPW, D), jnp.float32),
            pltpu.SemaphoreType.DMA,
            pltpu.SemaphoreType.DMA,
        ],
        compiler_params=pltpu.CompilerParams(use_tc_tiling_on_sc=True),
    )
    return run(idx, par, table2)
